# baseline (speedup 1.0000x reference)
import functools

import jax
import jax.numpy as jnp
from jax import lax
from jax.experimental import pallas as pl
from jax.experimental.pallas import tpu as pltpu
from jax.experimental.pallas import tpu_sc as plsc

B = 16384
EMB = 64
NC, NS, L = 1, 16, 16
NW = NC * NS
BPW = B // NW
GROUPS = BPW // L

_mesh = plsc.VectorSubcoreMesh(
    core_axis_name="c", subcore_axis_name="s", num_cores=NC, num_subcores=NS
)


@functools.partial(
    pl.kernel,
    out_type=jax.ShapeDtypeStruct((B * EMB,), jnp.float32),
    mesh=_mesh,
    scratch_types=[
        pltpu.VMEM((BPW * 2,), jnp.int32),
        pltpu.VMEM((16 * EMB,), jnp.float32),
        pltpu.VMEM((BPW * EMB,), jnp.float32),
    ],
    compiler_params=pltpu.CompilerParams(
        needs_layout_passes=False, use_tc_tiling_on_sc=False
    ),
)
def _encode(x_hbm, tab_hbm, out_hbm, xv, tabv, rows):
    wid = lax.axis_index("s") * NC + lax.axis_index("c")
    base = wid * BPW

    pltpu.sync_copy(x_hbm.at[pl.ds(base * 2, BPW * 2)], xv)
    pltpu.sync_copy(tab_hbm, tabv)

    lane = lax.iota(jnp.int32, L)
    lane2 = lane * 2
    lane64 = lane * EMB

    def group(g, carry):
        idx16 = plsc.load_gather(xv, [lane2 + 2 * L * g])
        s_i32 = plsc.load_gather(xv, [lane2 + (2 * L * g + 1)])
        s_f32 = s_i32.astype(jnp.float32) / 1000.0
        src64 = idx16 * EMB
        dst64 = lane64 + (L * EMB) * g
        for c in range(EMB - 1):
            vals = plsc.load_gather(tabv, [src64 + c])
            plsc.store_scatter(rows, [dst64 + c], vals)
        plsc.store_scatter(rows, [dst64 + (EMB - 1)], s_f32)
        return carry

    lax.fori_loop(0, GROUPS, group, 0)

    pltpu.sync_copy(rows, out_hbm.at[pl.ds(base * EMB, BPW * EMB)])


def kernel(x, type_embedding):
    tab = jnp.pad(type_embedding, ((0, 0), (0, 1)))
    out = _encode(x.reshape(-1).astype(jnp.int32), tab.reshape(-1))
    return out.reshape(B, EMB)


# speedup vs baseline: 2.1360x; 2.1360x over previous
import functools

import jax
import jax.numpy as jnp
from jax import lax
from jax.experimental import pallas as pl
from jax.experimental.pallas import tpu as pltpu
from jax.experimental.pallas import tpu_sc as plsc

B = 16384
EMB = 64
NC, NS, L = 2, 16, 16
NW = NC * NS
BPW = B // NW

_mesh = plsc.VectorSubcoreMesh(
    core_axis_name="c", subcore_axis_name="s", num_cores=NC, num_subcores=NS
)


@functools.partial(
    pl.kernel,
    out_type=jax.ShapeDtypeStruct((B * EMB,), jnp.float32),
    mesh=_mesh,
    scratch_types=[
        pltpu.VMEM((BPW * 2,), jnp.int32),
        pltpu.VMEM((16 * EMB,), jnp.float32),
        pltpu.VMEM((BPW * EMB,), jnp.float32),
    ],
    compiler_params=pltpu.CompilerParams(
        needs_layout_passes=False, use_tc_tiling_on_sc=False
    ),
)
def _encode(x_hbm, tab_hbm, out_hbm, xv, tabv, rows):
    wid = lax.axis_index("s") * NC + lax.axis_index("c")
    base = wid * BPW

    pltpu.sync_copy(x_hbm.at[pl.ds(base * 2, BPW * 2)], xv)
    pltpu.sync_copy(tab_hbm, tabv)

    last = lax.iota(jnp.int32, L) == (L - 1)

    def chunk(j, carry):
        v = xv[pl.ds(L * j, L)]
        for u in range(L // 2):
            t = v[2 * u]
            s = v[2 * u + 1].astype(jnp.float32) * (1.0 / 1000.0)
            src = t * EMB
            dst = ((L // 2) * j + u) * EMB
            for k in range(EMB // L - 1):
                rows[pl.ds(dst + k * L, L)] = tabv[pl.ds(src + k * L, L)]
            tail = tabv[pl.ds(src + EMB - L, L)]
            tail = jnp.where(last, jnp.full((L,), s, jnp.float32), tail)
            rows[pl.ds(dst + EMB - L, L)] = tail
        return carry

    lax.fori_loop(0, BPW // (L // 2), chunk, 0)

    pltpu.sync_copy(rows, out_hbm.at[pl.ds(base * EMB, BPW * EMB)])


def kernel(x, type_embedding):
    tab = jnp.pad(type_embedding, ((0, 0), (0, 1)))
    out = _encode(x.reshape(-1).astype(jnp.int32), tab.reshape(-1))
    return out.reshape(B, EMB)
